# Initial kernel scaffold; baseline (speedup 1.0000x reference)
import jax
import jax.numpy as jnp
from jax.experimental import pallas as pl


def kernel(hidden_states, gate_w, w1, w3, w2):
    raise NotImplementedError("write your pallas kernel here")


# speedup vs baseline: 2.5155x; 2.5155x over previous
import functools

import jax
import jax.numpy as jnp
from jax.experimental import pallas as pl
from jax.experimental.pallas import tpu as pltpu

E = 8
TOPK = 2
DMODEL = 1024
DFF = 512
NGROUPS = 2
GROUP_SIZE = E // NGROUPS
N_TOKENS = 2048


def _routing_kernel(x_ref, gw_ref, comb_ref):
    logits = jnp.dot(x_ref[...], gw_ref[...], preferred_element_type=jnp.float32)
    cols = [logits[:, i:i + 1] for i in range(E)]
    m0 = cols[0]
    for i in range(1, GROUP_SIZE):
        m0 = jnp.maximum(m0, cols[i])
    m1 = cols[GROUP_SIZE]
    for i in range(GROUP_SIZE + 1, E):
        m1 = jnp.maximum(m1, cols[i])
    mm = jnp.maximum(m0, m1)
    e0 = jnp.exp(m0 - mm)
    e1 = jnp.exp(m1 - mm)
    w0 = e0 / (e0 + e1)
    w1 = 1.0 - w0
    for g, (mg, wg) in enumerate(((m0, w0), (m1, w1))):
        seen = None
        for i in range(GROUP_SIZE):
            c = cols[g * GROUP_SIZE + i]
            hit = c >= mg
            sel = hit if seen is None else (hit & ~seen)
            seen = hit if seen is None else (seen | hit)
            comb_ref[g * GROUP_SIZE + i] = jnp.where(sel, wg, 0.0)


def _ffn_kernel(comb_ref, x_ref, w1_ref, w3_ref, w2_ref, out_ref):
    e = pl.program_id(0)
    x = x_ref[...]
    a = jax.lax.dot_general(x, w1_ref[0], (((1,), (1,)), ((), ())),
                            preferred_element_type=jnp.float32)
    b = jax.lax.dot_general(x, w3_ref[0], (((1,), (1,)), ((), ())),
                            preferred_element_type=jnp.float32)
    h = a * jax.nn.sigmoid(a) * b
    y = jax.lax.dot_general(h, w2_ref[0], (((1,), (1,)), ((), ())),
                            preferred_element_type=jnp.float32)
    contrib = comb_ref[0] * y

    @pl.when(e == 0)
    def _():
        out_ref[...] = contrib

    @pl.when(e > 0)
    def _():
        out_ref[...] += contrib


@jax.jit
def kernel(hidden_states, gate_w, w1, w3, w2):
    combine = pl.pallas_call(
        _routing_kernel,
        out_shape=jax.ShapeDtypeStruct((E, N_TOKENS, 1), jnp.float32),
    )(hidden_states, gate_w)

    out = pl.pallas_call(
        _ffn_kernel,
        grid=(E,),
        in_specs=[
            pl.BlockSpec((1, N_TOKENS, 1), lambda e: (e, 0, 0)),
            pl.BlockSpec((N_TOKENS, DMODEL), lambda e: (0, 0)),
            pl.BlockSpec((1, DFF, DMODEL), lambda e: (e, 0, 0)),
            pl.BlockSpec((1, DFF, DMODEL), lambda e: (e, 0, 0)),
            pl.BlockSpec((1, DMODEL, DFF), lambda e: (e, 0, 0)),
        ],
        out_specs=pl.BlockSpec((N_TOKENS, DMODEL), lambda e: (0, 0)),
        out_shape=jax.ShapeDtypeStruct((N_TOKENS, DMODEL), jnp.float32),
    )(combine, hidden_states, w1, w3, w2)
    return out
